# Initial kernel scaffold; baseline (speedup 1.0000x reference)
import jax
import jax.numpy as jnp
from jax.experimental import pallas as pl


def kernel(indices, table):
    raise NotImplementedError("write your pallas kernel here")


# speedup vs baseline: 2.7632x; 2.7632x over previous
import functools

import jax
import jax.numpy as jnp
from jax import lax
from jax.experimental import pallas as pl
from jax.experimental.pallas import tpu as pltpu
from jax.experimental.pallas import tpu_sc as plsc

_NC = 2
_NS = 16
_NW = _NC * _NS

_STREAM = 128
_K = 8
_CHUNK = _K * _STREAM


def _embed_body(idx_hbm, table_hbm, out_hbm, idx_v, rows_v, sem):
    D = table_hbm.shape[1]
    n_idx_rows = idx_hbm.shape[0]
    rows_per_w = n_idx_rows // _NW
    chunks_per_w = rows_per_w // _K

    wid = lax.axis_index("s") * _NC + lax.axis_index("c")
    base_row = wid * rows_per_w

    def chunk(g, carry):
        crow = base_row + g * _K
        pltpu.sync_copy(idx_hbm.at[pl.ds(crow, _K)], idx_v)
        copies = []
        for j in range(_K):
            copies.append(
                pltpu.async_copy(
                    table_hbm.at[idx_v.at[j]],
                    rows_v.at[pl.ds(j * _STREAM, _STREAM)],
                    sem,
                )
            )
        for c in copies:
            c.wait()
        pltpu.sync_copy(rows_v, out_hbm.at[pl.ds(crow * _STREAM, _CHUNK)])
        return carry

    lax.fori_loop(0, chunks_per_w, chunk, 0)


def kernel(indices, table):
    R, C = indices.shape
    V, D = table.shape
    B = R * C
    assert B % (_NW * _CHUNK) == 0

    idx2d = indices.reshape(B // _STREAM, _STREAM)

    mesh = plsc.VectorSubcoreMesh(core_axis_name="c", subcore_axis_name="s")
    embed = functools.partial(
        pl.kernel,
        out_type=jax.ShapeDtypeStruct((B, D), jnp.float32),
        mesh=mesh,
        scratch_types=[
            pltpu.VMEM((_K, _STREAM), jnp.int32),
            pltpu.VMEM((_CHUNK, D), jnp.float32),
            pltpu.SemaphoreType.DMA,
        ],
        compiler_params=pltpu.CompilerParams(use_tc_tiling_on_sc=False),
    )(_embed_body)

    out = embed(idx2d, table)
    return out.reshape(R, C, D)


# speedup vs baseline: 4.7421x; 1.7162x over previous
import functools

import jax
import jax.numpy as jnp
from jax import lax
from jax.experimental import pallas as pl
from jax.experimental.pallas import tpu as pltpu
from jax.experimental.pallas import tpu_sc as plsc

_NC = 2
_NS = 16
_NW = _NC * _NS

_STREAM = 128
_K = 8
_CHUNK = _K * _STREAM


def _embed_body(idx_hbm, table_hbm, out_hbm, table_v, idx_v, rows_v, sem):
    D = table_hbm.shape[1]
    n_idx_rows = idx_hbm.shape[0]
    rows_per_w = n_idx_rows // _NW
    chunks_per_w = rows_per_w // _K

    wid = lax.axis_index("s") * _NC + lax.axis_index("c")
    base_row = wid * rows_per_w

    @pl.when(lax.axis_index("s") == 0)
    def _():
        tmp = rows_v.at[pl.ds(0, table_hbm.shape[0])]
        pltpu.sync_copy(table_hbm, tmp)
        pltpu.sync_copy(tmp, table_v)

    plsc.subcore_barrier()

    def chunk(g, carry):
        crow = base_row + g * _K
        pltpu.sync_copy(idx_hbm.at[pl.ds(crow, _K)], idx_v)
        copies = []
        for j in range(_K):
            copies.append(
                pltpu.async_copy(
                    table_v.at[idx_v.at[j]],
                    rows_v.at[pl.ds(j * _STREAM, _STREAM)],
                    sem,
                )
            )
        for c in copies:
            c.wait()
        pltpu.sync_copy(rows_v, out_hbm.at[pl.ds(crow * _STREAM, _CHUNK)])
        return carry

    lax.fori_loop(0, chunks_per_w, chunk, 0)


def kernel(indices, table):
    R, C = indices.shape
    V, D = table.shape
    B = R * C
    assert B % (_NW * _CHUNK) == 0

    idx2d = indices.reshape(B // _STREAM, _STREAM)

    mesh = plsc.VectorSubcoreMesh(core_axis_name="c", subcore_axis_name="s")
    embed = functools.partial(
        pl.kernel,
        out_type=jax.ShapeDtypeStruct((B, D), jnp.float32),
        mesh=mesh,
        scratch_types=[
            pltpu.VMEM_SHARED((V, D), jnp.float32),
            pltpu.VMEM((_K, _STREAM), jnp.int32),
            pltpu.VMEM((_CHUNK, D), jnp.float32),
            pltpu.SemaphoreType.DMA,
        ],
        compiler_params=pltpu.CompilerParams(use_tc_tiling_on_sc=False),
    )(_embed_body)

    out = embed(idx2d, table)
    return out.reshape(R, C, D)


# speedup vs baseline: 4.7428x; 1.0001x over previous
import functools

import jax
import jax.numpy as jnp
from jax import lax
from jax.experimental import pallas as pl
from jax.experimental.pallas import tpu as pltpu
from jax.experimental.pallas import tpu_sc as plsc

_NC = 2
_NS = 16
_NW = _NC * _NS

_CHUNK = 1024


def _embed_body(idx_hbm, table_hbm, out_hbm, table_s, idx_v, rows_v, gsem):
    D = table_hbm.shape[1]
    V = table_hbm.shape[0]
    B = idx_hbm.shape[0]
    per_w = B // _NW
    chunks_per_w = per_w // _CHUNK

    wid = lax.axis_index("s") * _NC + lax.axis_index("c")
    base = wid * per_w

    @pl.when(lax.axis_index("s") == 0)
    def _():
        tmp = rows_v.at[pl.ds(0, V)]
        pltpu.sync_copy(table_hbm, tmp)
        pltpu.sync_copy(tmp, table_s)

    plsc.subcore_barrier()

    def chunk(g, carry):
        cbase = base + g * _CHUNK
        pltpu.sync_copy(idx_hbm.at[pl.ds(cbase, _CHUNK)], idx_v)
        pltpu.async_copy(table_s.at[idx_v], rows_v, gsem).wait()
        pltpu.sync_copy(rows_v, out_hbm.at[pl.ds(cbase, _CHUNK)])
        return carry

    lax.fori_loop(0, chunks_per_w, chunk, 0)


def kernel(indices, table):
    R, C = indices.shape
    V, D = table.shape
    B = R * C
    assert B % (_NW * _CHUNK) == 0

    idx_flat = indices.reshape(B)

    mesh = plsc.VectorSubcoreMesh(core_axis_name="c", subcore_axis_name="s")
    embed = functools.partial(
        pl.kernel,
        out_type=jax.ShapeDtypeStruct((B, D), jnp.float32),
        mesh=mesh,
        scratch_types=[
            pltpu.VMEM_SHARED((V, D), jnp.float32),
            pltpu.VMEM((_CHUNK,), jnp.int32),
            pltpu.VMEM((_CHUNK, D), jnp.float32),
            pltpu.SemaphoreType.DMA,
        ],
        compiler_params=pltpu.CompilerParams(use_tc_tiling_on_sc=False),
    )(_embed_body)

    out = embed(idx_flat, table)
    return out.reshape(R, C, D)


# speedup vs baseline: 5.1554x; 1.0870x over previous
import functools

import jax
import jax.numpy as jnp
from jax import lax
from jax.experimental import pallas as pl
from jax.experimental.pallas import tpu as pltpu
from jax.experimental.pallas import tpu_sc as plsc

_NC = 2
_NS = 16
_NW = _NC * _NS

_CHUNK = 1024


_SUPER = 10


def _embed_body(idx_hbm, table_hbm, out_hbm, table_s,
                idx_super, rows0, rows1, gsem, wsem):
    D = table_hbm.shape[1]
    V = table_hbm.shape[0]
    B = idx_hbm.shape[0]
    per_w = B // _NW
    chunks_per_w = per_w // _CHUNK
    n_super = chunks_per_w // _SUPER

    wid = lax.axis_index("s") * _NC + lax.axis_index("c")
    base = wid * per_w

    @pl.when(lax.axis_index("s") == 0)
    def _():
        tmp = rows0.at[pl.ds(0, V)]
        pltpu.sync_copy(table_hbm, tmp)
        pltpu.sync_copy(tmp, table_s)

    plsc.subcore_barrier()

    rowsb = (rows0, rows1)

    def super_chunk(s, carry):
        sbase = base + s * _SUPER * _CHUNK
        pltpu.sync_copy(idx_hbm.at[pl.ds(sbase, _SUPER * _CHUNK)], idx_super)

        def wb_start(c):
            return pltpu.async_copy(
                rowsb[c % 2], out_hbm.at[pl.ds(sbase + c * _CHUNK, _CHUNK)],
                wsem)

        wh = None
        for c in range(_SUPER):
            p = c % 2
            pltpu.async_copy(
                table_s.at[idx_super.at[pl.ds(c * _CHUNK, _CHUNK)]],
                rowsb[p], gsem).wait()
            if wh is not None:
                wh.wait()
            wh = wb_start(c)
        wh.wait()
        return carry

    lax.fori_loop(0, n_super, super_chunk, 0)


def kernel(indices, table):
    R, C = indices.shape
    V, D = table.shape
    B = R * C
    assert B % (_NW * _CHUNK * _SUPER) == 0

    idx_flat = indices.reshape(B)

    mesh = plsc.VectorSubcoreMesh(core_axis_name="c", subcore_axis_name="s")
    embed = functools.partial(
        pl.kernel,
        out_type=jax.ShapeDtypeStruct((B, D), jnp.float32),
        mesh=mesh,
        scratch_types=[
            pltpu.VMEM_SHARED((V, D), jnp.float32),
            pltpu.VMEM((_SUPER * _CHUNK,), jnp.int32),
            pltpu.VMEM((_CHUNK, D), jnp.float32),
            pltpu.VMEM((_CHUNK, D), jnp.float32),
            pltpu.SemaphoreType.DMA,
            pltpu.SemaphoreType.DMA,
        ],
        compiler_params=pltpu.CompilerParams(use_tc_tiling_on_sc=False),
    )(_embed_body)

    out = embed(idx_flat, table)
    return out.reshape(R, C, D)
